# baseline (speedup 1.0000x reference)
import functools

import jax
import jax.numpy as jnp
from jax import lax
from jax.experimental import pallas as pl
from jax.experimental.pallas import tpu as pltpu
from jax.experimental.pallas import tpu_sc as plsc

N = 10000
E = 320000
D = 128

NC = 2
NS = 16
NW = NC * NS
E_PER_W = E // NW
CHUNK = 128
REAL = 125
N_CHUNKS = E_PER_W // REAL
E_PAD = N_CHUNKS * CHUNK
N_PAIRS = N_CHUNKS // 2
WB_TILES = 10
WB_ROWS = N // WB_TILES


def _sc_spmm(packed3, val3, embeds, zeros_blk):
    mesh = plsc.VectorSubcoreMesh(core_axis_name="c", subcore_axis_name="s")

    @functools.partial(
        pl.kernel,
        out_type=jax.ShapeDtypeStruct((NC, N, D), jnp.float32),
        mesh=mesh,
        scratch_types=[
            pltpu.VMEM_SHARED((N, D), jnp.float32),
            pltpu.VMEM((E_PAD,), jnp.int32),
            pltpu.VMEM((CHUNK,), jnp.int32),
            pltpu.VMEM((CHUNK,), jnp.int32),
            pltpu.VMEM((CHUNK,), jnp.int32),
            pltpu.VMEM((CHUNK,), jnp.int32),
            pltpu.VMEM((CHUNK,), jnp.float32),
            pltpu.VMEM((CHUNK,), jnp.float32),
            pltpu.VMEM((CHUNK, D), jnp.float32),
            pltpu.VMEM((CHUNK, D), jnp.float32),
            pltpu.SemaphoreType.DMA,
            pltpu.SemaphoreType.DMA,
            pltpu.SemaphoreType.DMA,
            pltpu.SemaphoreType.DMA,
        ],
    )
    def k(packed_h, val_h, emb_h, zero_h, out_h,
          acc, packed, colb0, colb1, rowb0, rowb1, valf0, valf1, buf0, buf1,
          gs0, gs1, ss0, ss1):
        cid = lax.axis_index("c")
        sid = lax.axis_index("s")
        wid = cid * NS + sid

        pltpu.sync_copy(packed_h.at[wid], packed)

        def unpack(ci, colb, rowb):
            for g in range(CHUNK // 16):
                sl = pl.ds(g * 16, 16)
                p = packed[pl.ds(ci * CHUNK + g * 16, 16)]
                colb[sl] = lax.bitwise_and(p, 0xFFFF)
                rowb[sl] = lax.shift_right_logical(p, 16)

        def gather_start(ci, buf, colb, valf, sem):
            pltpu.async_copy(emb_h.at[colb], buf, sem)
            pltpu.async_copy(val_h.at[wid, pl.ds(ci * CHUNK, CHUNK)], valf, sem)

        def gather_wait(ci, buf, colb, valf, sem):
            pltpu.make_async_copy(emb_h.at[colb], buf, sem).wait()
            pltpu.make_async_copy(
                val_h.at[wid, pl.ds(ci * CHUNK, CHUNK)], valf, sem).wait()

        def scatter_start(buf, rowb, sem):
            pltpu.async_copy(buf, acc.at[rowb], sem, add=True)

        def scatter_wait(buf, rowb, sem):
            pltpu.make_async_copy(buf, acc.at[rowb], sem).wait()

        def scale(buf, valf):
            def g_body(g, carry):
                vv = valf[pl.ds(g * 16, 16)]
                for t in range(16):
                    v = vv[t]
                    e = g * 16 + t
                    for j in range(D // 16):
                        sl = pl.ds(j * 16, 16)
                        buf[e, sl] = buf[e, sl] * v
                return carry

            lax.fori_loop(0, CHUNK // 16, g_body, 0)

        unpack(0, colb0, rowb0)
        gather_start(0, buf0, colb0, valf0, gs0)
        unpack(1, colb1, rowb1)
        gather_start(1, buf1, colb1, valf1, gs1)

        @pl.when(sid < WB_TILES)
        def _():
            pltpu.sync_copy(zero_h, acc.at[pl.ds(sid * WB_ROWS, WB_ROWS)])

        plsc.subcore_barrier()

        def pair_body(i, carry):
            c0 = 2 * i
            c1 = 2 * i + 1
            gather_wait(c0, buf0, colb0, valf0, gs0)
            scale(buf0, valf0)
            scatter_start(buf0, rowb0, ss0)

            gather_wait(c1, buf1, colb1, valf1, gs1)
            scale(buf1, valf1)
            scatter_start(buf1, rowb1, ss1)

            @pl.when(i < N_PAIRS - 1)
            def _():
                scatter_wait(buf0, rowb0, ss0)
                unpack(c0 + 2, colb0, rowb0)
                gather_start(c0 + 2, buf0, colb0, valf0, gs0)

                scatter_wait(buf1, rowb1, ss1)
                unpack(c1 + 2, colb1, rowb1)
                gather_start(c1 + 2, buf1, colb1, valf1, gs1)

            return carry

        lax.fori_loop(0, N_PAIRS, pair_body, 0)

        scatter_wait(buf0, rowb0, ss0)
        scatter_wait(buf1, rowb1, ss1)

        plsc.subcore_barrier()

        @pl.when(sid < WB_TILES)
        def _():
            sl = pl.ds(sid * WB_ROWS, WB_ROWS)
            pltpu.sync_copy(acc.at[sl], out_h.at[cid, sl])

    return k(packed3, val3, embeds, zeros_blk)


def _tc_add(partials):
    def body(p_ref, o_ref):
        o_ref[...] = p_ref[0] + p_ref[1]

    return pl.pallas_call(
        body,
        out_shape=jax.ShapeDtypeStruct((N, D), jnp.float32),
        grid=(10,),
        in_specs=[pl.BlockSpec((NC, N // 10, D), lambda i: (0, i, 0))],
        out_specs=pl.BlockSpec((N // 10, D), lambda i: (i, 0)),
    )(partials)


def kernel(edge_index, edge_values, embeds):
    row = edge_index[0].astype(jnp.int32).reshape(NW, N_CHUNKS, REAL)
    col = edge_index[1].astype(jnp.int32).reshape(NW, N_CHUNKS, REAL)
    val = edge_values.reshape(NW, N_CHUNKS, REAL)
    pad = ((0, 0), (0, 0), (0, CHUNK - REAL))
    packed3 = ((jnp.pad(row, pad) << 16) | jnp.pad(col, pad)).reshape(NW, E_PAD)
    val3 = jnp.pad(val, pad).reshape(NW, E_PAD)
    zeros_blk = jnp.zeros((WB_ROWS, D), jnp.float32)
    partials = _sc_spmm(packed3, val3, embeds, zeros_blk)
    return _tc_add(partials)


# speedup vs baseline: 2.1288x; 2.1288x over previous
import functools

import jax
import jax.numpy as jnp
from jax import lax
from jax.experimental import pallas as pl
from jax.experimental.pallas import tpu as pltpu
from jax.experimental.pallas import tpu_sc as plsc

N = 10000
E = 320000
D = 128

NC = 2
NS = 16
NW = NC * NS
E_PER_W = E // NW
CHUNK = 128
REAL = 125
N_CHUNKS = E_PER_W // REAL
E_PAD = N_CHUNKS * CHUNK
N_PAIRS = N_CHUNKS // 2
WB_TILES = 10
WB_ROWS = N // WB_TILES


def _sc_spmm(packed3, val3, embeds, zeros_blk):
    mesh = plsc.VectorSubcoreMesh(core_axis_name="c", subcore_axis_name="s")

    @functools.partial(
        pl.kernel,
        out_type=jax.ShapeDtypeStruct((NC, N, D), jnp.float32),
        mesh=mesh,
        scratch_types=[
            pltpu.VMEM_SHARED((N, D), jnp.float32),
            pltpu.VMEM((E_PAD,), jnp.int32),
            pltpu.VMEM((CHUNK,), jnp.int32),
            pltpu.VMEM((CHUNK,), jnp.int32),
            pltpu.VMEM((CHUNK,), jnp.int32),
            pltpu.VMEM((CHUNK,), jnp.int32),
            pltpu.VMEM((CHUNK,), jnp.float32),
            pltpu.VMEM((CHUNK,), jnp.float32),
            pltpu.VMEM((CHUNK, D), jnp.float32),
            pltpu.VMEM((CHUNK, D), jnp.float32),
            pltpu.SemaphoreType.DMA,
            pltpu.SemaphoreType.DMA,
            pltpu.SemaphoreType.DMA,
            pltpu.SemaphoreType.DMA,
        ],
    )
    def k(packed_h, val_h, emb_h, zero_h, out_h,
          acc, packed, colb0, colb1, rowb0, rowb1, valf0, valf1, buf0, buf1,
          gs0, gs1, ss0, ss1):
        cid = lax.axis_index("c")
        sid = lax.axis_index("s")
        wid = cid * NS + sid

        pltpu.sync_copy(packed_h.at[wid], packed)

        def unpack(ci, colb, rowb):
            for g in range(CHUNK // 16):
                sl = pl.ds(g * 16, 16)
                p = packed[pl.ds(ci * CHUNK + g * 16, 16)]
                colb[sl] = lax.bitwise_and(p, 0xFFFF)
                rowb[sl] = lax.shift_right_logical(p, 16)

        def gather_start(ci, buf, colb, valf, sem):
            pltpu.async_copy(emb_h.at[colb], buf, sem)
            pltpu.async_copy(val_h.at[wid, pl.ds(ci * CHUNK, CHUNK)], valf, sem)

        def gather_wait(ci, buf, colb, valf, sem):
            pltpu.make_async_copy(emb_h.at[colb], buf, sem).wait()
            pltpu.make_async_copy(
                val_h.at[wid, pl.ds(ci * CHUNK, CHUNK)], valf, sem).wait()

        def scatter_start(buf, rowb, sem):
            pltpu.async_copy(buf, acc.at[rowb], sem, add=True)

        def scatter_wait(buf, rowb, sem):
            pltpu.make_async_copy(buf, acc.at[rowb], sem).wait()

        def scale(buf, valf):
            def g_body(g, carry):
                vv = valf[pl.ds(g * 16, 16)]
                for t in range(16):
                    v = vv[t]
                    e = g * 16 + t
                    for j in range(D // 16):
                        sl = pl.ds(j * 16, 16)
                        buf[e, sl] = buf[e, sl] * v
                return carry

            lax.fori_loop(0, CHUNK // 16, g_body, 0)

        unpack(0, colb0, rowb0)
        gather_start(0, buf0, colb0, valf0, gs0)
        unpack(1, colb1, rowb1)
        gather_start(1, buf1, colb1, valf1, gs1)

        @pl.when(sid < WB_TILES)
        def _():
            pltpu.sync_copy(zero_h, acc.at[pl.ds(sid * WB_ROWS, WB_ROWS)])

        plsc.subcore_barrier()

        def pair_body(i, carry):
            c0 = 2 * i
            c1 = 2 * i + 1
            gather_wait(c0, buf0, colb0, valf0, gs0)
            scale(buf0, valf0)
            scatter_start(buf0, rowb0, ss0)

            gather_wait(c1, buf1, colb1, valf1, gs1)
            scale(buf1, valf1)
            scatter_start(buf1, rowb1, ss1)

            @pl.when(i < N_PAIRS - 1)
            def _():
                scatter_wait(buf0, rowb0, ss0)
                unpack(c0 + 2, colb0, rowb0)
                gather_start(c0 + 2, buf0, colb0, valf0, gs0)

                scatter_wait(buf1, rowb1, ss1)
                unpack(c1 + 2, colb1, rowb1)
                gather_start(c1 + 2, buf1, colb1, valf1, gs1)

            return carry

        lax.fori_loop(0, N_PAIRS, pair_body, 0)

        scatter_wait(buf0, rowb0, ss0)
        scatter_wait(buf1, rowb1, ss1)

        plsc.subcore_barrier()

        @pl.when(sid < WB_TILES)
        def _():
            sl = pl.ds(sid * WB_ROWS, WB_ROWS)
            pltpu.sync_copy(acc.at[sl], out_h.at[cid, sl])

    return k(packed3, val3, embeds, zeros_blk)


def _tc_add(partials):
    def body(p_ref, o_ref):
        o_ref[...] = p_ref[0] + p_ref[1]

    return pl.pallas_call(
        body,
        out_shape=jax.ShapeDtypeStruct((N, D), jnp.float32),
        grid=(10,),
        in_specs=[pl.BlockSpec((NC, N // 10, D), lambda i: (0, i, 0))],
        out_specs=pl.BlockSpec((N // 10, D), lambda i: (i, 0)),
    )(partials)


def kernel(edge_index, edge_values, embeds):
    row = edge_index[0].astype(jnp.int32).reshape(NW, N_CHUNKS, REAL)
    col = edge_index[1].astype(jnp.int32).reshape(NW, N_CHUNKS, REAL)
    val = edge_values.reshape(NW, N_CHUNKS, REAL)
    pad = ((0, 0), (0, 0), (0, CHUNK - REAL))
    spread = (jnp.arange(NW * N_CHUNKS * (CHUNK - REAL), dtype=jnp.int32)
              * 677 % N).reshape(NW, N_CHUNKS, CHUNK - REAL)
    rowp = jnp.concatenate([row, spread], axis=2)
    colp = jnp.concatenate([col, spread], axis=2)
    packed3 = ((rowp << 16) | colp).reshape(NW, E_PAD)
    val3 = jnp.pad(val, pad).reshape(NW, E_PAD)
    zeros_blk = jnp.zeros((WB_ROWS, D), jnp.float32)
    partials = _sc_spmm(packed3, val3, embeds, zeros_blk)
    return _tc_add(partials)
